# baseline (speedup 1.0000x reference)
import functools

import jax
import jax.numpy as jnp
from jax import lax
from jax.experimental import pallas as pl
from jax.experimental.pallas import tpu as pltpu
from jax.experimental.pallas import tpu_sc as plsc

BATCH = 16384
EMBED_DIM = 64
GROUP = 8
QUADS = BATCH // GROUP
QUAD_DIM = GROUP * EMBED_DIM

_info = plsc.get_sparse_core_info()
_NW = _info.num_cores * _info.num_subcores
_Q_PER_W = QUADS // _NW
_R_PER_W = BATCH // _NW
_LANES = 16


@functools.partial(
    pl.kernel,
    mesh=plsc.VectorSubcoreMesh(core_axis_name="c", subcore_axis_name="s"),
    out_type=jax.ShapeDtypeStruct((BATCH * EMBED_DIM,), jnp.float32),
    scratch_types=[
        pltpu.VMEM((_R_PER_W,), jnp.int32),
        pltpu.VMEM((_Q_PER_W,), jnp.int32),
        pltpu.VMEM((_Q_PER_W, QUAD_DIM), jnp.float32),
        pltpu.SemaphoreType.DMA,
    ],
    compiler_params=pltpu.CompilerParams(
        needs_layout_passes=False, use_tc_tiling_on_sc=False
    ),
)
def _quad_lookup(table_hbm, role_hbm, out_hbm, role_v, quad_v, rows_v, sem):
    wid = lax.axis_index("s") * _info.num_cores + lax.axis_index("c")
    pltpu.sync_copy(role_hbm.at[pl.ds(wid * _R_PER_W, _R_PER_W)], role_v)

    lane = lax.iota(jnp.int32, _LANES)
    for k in range(_Q_PER_W // _LANES):
        base = GROUP * _LANES * k
        q = plsc.load_gather(role_v, [base + GROUP * lane])
        for d in range(1, GROUP):
            q = 2 * q + plsc.load_gather(role_v, [base + GROUP * lane + d])
        quad_v[pl.ds(_LANES * k, _LANES)] = q

    pltpu.async_copy(table_hbm.at[quad_v], rows_v, sem).wait()
    base = wid * _Q_PER_W * QUAD_DIM
    writes = [
        pltpu.async_copy(
            rows_v.at[i], out_hbm.at[pl.ds(base + i * QUAD_DIM, QUAD_DIM)], sem
        )
        for i in range(_Q_PER_W)
    ]
    for w in writes:
        w.wait()


def kernel(obs, role, W_role):
    del obs
    t = W_role
    nrows = 1 << GROUP
    cols = []
    for d in range(GROUP):
        rep = 1 << (GROUP - 1 - d)
        cols.append(jnp.tile(jnp.repeat(t, rep, axis=0), (nrows // (2 * rep), 1)))
    table_g = jnp.concatenate(cols, axis=1)
    return _quad_lookup(table_g, role).reshape(BATCH, EMBED_DIM)
